# Initial kernel scaffold; baseline (speedup 1.0000x reference)
import jax
import jax.numpy as jnp
from jax.experimental import pallas as pl


def kernel(x, edge_index, batch, W1, att_src1, att_dst1, b1, W2, att_src2, att_dst2, b2, Wm1, bm1, Wm2, bm2):
    raise NotImplementedError("write your pallas kernel here")


# speedup vs baseline: 3.3145x; 3.3145x over previous
import functools

import jax
import jax.numpy as jnp
from jax.experimental import pallas as pl

_HEADS = 12
_CH = 768
_G = 128


def _mm_kernel(x_ref, w_ref, o_ref):
    o_ref[...] = jnp.dot(x_ref[...], w_ref[...],
                         preferred_element_type=jnp.float32)


def _pallas_matmul(x, w, bm=512, bn=768):
    m, k = x.shape
    k2, n = w.shape
    assert k == k2
    bm = min(bm, m)
    bn = min(bn, n)
    mp = ((m + bm - 1) // bm) * bm
    np_ = ((n + bn - 1) // bn) * bn
    xp = jnp.pad(x, ((0, mp - m), (0, 0))) if mp != m else x
    wp = jnp.pad(w, ((0, 0), (0, np_ - n))) if np_ != n else w
    out = pl.pallas_call(
        _mm_kernel,
        grid=(mp // bm, np_ // bn),
        in_specs=[
            pl.BlockSpec((bm, k), lambda i, j: (i, 0)),
            pl.BlockSpec((k, bn), lambda i, j: (0, j)),
        ],
        out_specs=pl.BlockSpec((bm, bn), lambda i, j: (i, j)),
        out_shape=jax.ShapeDtypeStruct((mp, np_), jnp.float32),
    )(xp, wp)
    return out[:m, :n]


def _gat_layer(h0, src, dst, W, att_src, att_dst, bias):
    n = h0.shape[0]
    Wr = W.reshape(_CH, _HEADS, _CH)
    A_src = jnp.einsum('khc,hc->kh', Wr, att_src[0])
    A_dst = jnp.einsum('khc,hc->kh', Wr, att_dst[0])
    A = jnp.concatenate([A_src, A_dst], axis=1)
    scores = _pallas_matmul(h0, A)
    a_src = scores[:, :_HEADS]
    a_dst = scores[:, _HEADS:]

    h = _pallas_matmul(h0, W)

    alpha = a_src[src] + a_dst[dst]
    alpha = jax.nn.leaky_relu(alpha, negative_slope=0.2)
    amax = jax.ops.segment_max(alpha, dst, num_segments=n,
                               indices_are_sorted=True)
    ex = jnp.exp(alpha - amax[dst])
    denom = jax.ops.segment_sum(ex, dst, num_segments=n,
                                indices_are_sorted=True)
    attn = ex / (denom[dst] + 1e-16)

    hr = h.reshape(n, _HEADS, _CH)
    msg = jnp.einsum('eh,ehc->ec', attn, hr[src])
    agg = jax.ops.segment_sum(msg, dst, num_segments=n,
                              indices_are_sorted=True) / _HEADS
    return agg + bias


def kernel(x, edge_index, batch, W1, att_src1, att_dst1, b1,
           W2, att_src2, att_dst2, b2, Wm1, bm1, Wm2, bm2):
    src = edge_index[0]
    dst = edge_index[1]
    order = jnp.argsort(dst)
    src = src[order]
    dst = dst[order]

    h = _gat_layer(x, src, dst, W1, att_src1, att_dst1, b1)
    h = _gat_layer(h, src, dst, W2, att_src2, att_dst2, b2)

    ssum = jax.ops.segment_sum(h, batch, num_segments=_G,
                               indices_are_sorted=True)
    cnt = jax.ops.segment_sum(jnp.ones((h.shape[0], 1), h.dtype), batch,
                              num_segments=_G, indices_are_sorted=True)
    g = ssum / jnp.maximum(cnt, 1.0)

    z = jax.nn.relu(_pallas_matmul(g, Wm1) + bm1)
    return _pallas_matmul(z, Wm2) + bm2


# speedup vs baseline: 3.3153x; 1.0002x over previous
import functools

import jax
import jax.numpy as jnp
from jax.experimental import pallas as pl

_HEADS = 12
_CH = 768
_G = 128


def _mm_kernel(x_ref, w_ref, o_ref):
    o_ref[...] = jnp.dot(x_ref[...], w_ref[...],
                         preferred_element_type=jnp.float32)


def _pallas_matmul(x, w, bm=512, bn=768):
    m, k = x.shape
    k2, n = w.shape
    assert k == k2
    bm = min(bm, m)
    bn = min(bn, n)
    mp = ((m + bm - 1) // bm) * bm
    np_ = ((n + bn - 1) // bn) * bn
    xp = jnp.pad(x, ((0, mp - m), (0, 0))) if mp != m else x
    wp = jnp.pad(w, ((0, 0), (0, np_ - n))) if np_ != n else w
    out = pl.pallas_call(
        _mm_kernel,
        grid=(mp // bm, np_ // bn),
        in_specs=[
            pl.BlockSpec((bm, k), lambda i, j: (i, 0)),
            pl.BlockSpec((k, bn), lambda i, j: (0, j)),
        ],
        out_specs=pl.BlockSpec((bm, bn), lambda i, j: (i, j)),
        out_shape=jax.ShapeDtypeStruct((mp, np_), jnp.float32),
    )(xp, wp)
    return out[:m, :n]


def _gat_layer(h0, src, dst, W, att_src, att_dst, bias):
    n = h0.shape[0]
    Wr = W.reshape(_CH, _HEADS, _CH)
    A_src = jnp.einsum('khc,hc->kh', Wr, att_src[0])
    A_dst = jnp.einsum('khc,hc->kh', Wr, att_dst[0])
    A = jnp.concatenate([A_src, A_dst], axis=1)
    scores = _pallas_matmul(h0, A)
    a_src = scores[:, :_HEADS]
    a_dst = scores[:, _HEADS:]

    h = _pallas_matmul(h0, W)

    alpha = a_src[src] + a_dst[dst]
    alpha = jax.nn.leaky_relu(alpha, negative_slope=0.2)
    amax = jax.ops.segment_max(alpha, dst, num_segments=n,
                               indices_are_sorted=True)
    ex = jnp.exp(alpha - amax[dst])
    denom = jax.ops.segment_sum(ex, dst, num_segments=n,
                                indices_are_sorted=True)
    attn = ex / (denom[dst] + 1e-16)

    hr = h.astype(jnp.bfloat16).reshape(n, _HEADS, _CH)
    msg = jnp.einsum('eh,ehc->ec', attn, hr[src],
                     preferred_element_type=jnp.float32)
    agg = jax.ops.segment_sum(msg, dst, num_segments=n,
                              indices_are_sorted=True) / _HEADS
    return agg + bias


def kernel(x, edge_index, batch, W1, att_src1, att_dst1, b1,
           W2, att_src2, att_dst2, b2, Wm1, bm1, Wm2, bm2):
    src = edge_index[0]
    dst = edge_index[1]
    order = jnp.argsort(dst)
    src = src[order]
    dst = dst[order]

    h = _gat_layer(x, src, dst, W1, att_src1, att_dst1, b1)
    h = _gat_layer(h, src, dst, W2, att_src2, att_dst2, b2)

    ssum = jax.ops.segment_sum(h, batch, num_segments=_G,
                               indices_are_sorted=True)
    cnt = jax.ops.segment_sum(jnp.ones((h.shape[0], 1), h.dtype), batch,
                              num_segments=_G, indices_are_sorted=True)
    g = ssum / jnp.maximum(cnt, 1.0)

    z = jax.nn.relu(_pallas_matmul(g, Wm1) + bm1)
    return _pallas_matmul(z, Wm2) + bm2
